# baseline (speedup 1.0000x reference)
import functools

import jax
import jax.numpy as jnp
from jax import lax
from jax.experimental import pallas as pl
from jax.experimental.pallas import tpu as pltpu
from jax.experimental.pallas import tpu_sc as plsc

N = 10000
E = 320000
D = 128
NC = 2
NS = 16
NW = NC * NS
CH = 128
NCHUNK = 2 * (-(-E // (NW * CH * 2)))
EPT = NCHUNK * CH
E_PAD = EPT * NW
N_PAD = 10240
NPT = N_PAD // NS
RB = 128

_sc_mesh = plsc.VectorSubcoreMesh(core_axis_name="c", subcore_axis_name="s")


def _deg_body(col_hbm, zeros_hbm, out_hbm, cidx, ones, tmp, shared):
    cid = lax.axis_index("c")
    sid = lax.axis_index("s")
    wid = sid * NC + cid
    pltpu.sync_copy(zeros_hbm.at[pl.ds(sid * NPT, NPT)], tmp)
    pltpu.sync_copy(tmp, shared.at[pl.ds(sid * NPT, NPT)])
    for i in range(CH // 16):
        ones[pl.ds(i * 16, 16)] = jnp.ones((16,), jnp.float32)
    pltpu.sync_copy(col_hbm.at[wid], cidx)
    plsc.subcore_barrier()

    def body(j, carry):
        pltpu.sync_copy(ones, shared.at[cidx.at[j]], add=True)
        return carry

    lax.fori_loop(0, NCHUNK, body, 0)
    plsc.subcore_barrier()
    pltpu.sync_copy(shared.at[pl.ds(sid * NPT, NPT)], tmp)
    pltpu.sync_copy(tmp, out_hbm.at[cid, pl.ds(sid * NPT, NPT)])


_deg_kernel = functools.partial(
    pl.kernel,
    out_type=jax.ShapeDtypeStruct((NC, N_PAD), jnp.float32),
    mesh=_sc_mesh,
    scratch_types=[
        pltpu.VMEM((NCHUNK, CH), jnp.int32),
        pltpu.VMEM((CH,), jnp.float32),
        pltpu.VMEM((NPT,), jnp.float32),
        pltpu.VMEM_SHARED((N_PAD,), jnp.float32),
    ],
)(_deg_body)


def _dis_col(dp_ref):
    pid = pl.program_id(0)
    deg_row = dp_ref[0, pid, :] + dp_ref[1, pid, :] + 1.0
    dis_row = lax.rsqrt(deg_row)
    m = (lax.broadcasted_iota(jnp.int32, (RB, RB), 0)
         == lax.broadcasted_iota(jnp.int32, (RB, RB), 1))
    return jnp.sum(jnp.where(m, dis_row[None, :], 0.0), axis=1,
                   keepdims=True)


def _mm_body(x_ref, w_ref, dp_ref, g_ref):
    h = jnp.dot(x_ref[...], w_ref[...], preferred_element_type=jnp.float32)
    g_ref[...] = h * _dis_col(dp_ref)


_mm_kernel = pl.pallas_call(
    _mm_body,
    grid=(N_PAD // RB,),
    in_specs=[
        pl.BlockSpec((RB, D), lambda i: (i, 0)),
        pl.BlockSpec((D, D), lambda i: (0, 0)),
        pl.BlockSpec((2, N_PAD // 128, 128), lambda i: (0, 0, 0)),
    ],
    out_specs=pl.BlockSpec((RB, D), lambda i: (i, 0)),
    out_shape=jax.ShapeDtypeStruct((N_PAD, D), jnp.float32),
)


def _agg_body(row_hbm, col_hbm, g_hbm, zeros_hbm, out_hbm,
              ridx, cidx, rows0, rows1, acc, sg0, sg1, si0, si1):
    cid = lax.axis_index("c")
    sid = lax.axis_index("s")
    wid = sid * NC + cid

    def zbody(t, carry):
        r0 = sid * NPT + t * RB
        pltpu.sync_copy(zeros_hbm.at[pl.ds(r0, RB)], rows0)
        pltpu.sync_copy(rows0, acc.at[pl.ds(r0, RB)])
        return carry

    lax.fori_loop(0, NPT // RB, zbody, 0)
    pltpu.sync_copy(col_hbm.at[wid], cidx)
    plsc.subcore_barrier()

    pltpu.sync_copy(row_hbm.at[wid, 0], ridx.at[0])
    pltpu.async_copy(g_hbm.at[ridx.at[0]], rows0, sg0)
    pltpu.async_copy(row_hbm.at[wid, 1], ridx.at[1], si1)

    def body(t, carry):
        j0 = 2 * t
        pltpu.make_async_copy(g_hbm.at[ridx.at[0]], rows0, sg0).wait()

        @pl.when(j0 + 2 < NCHUNK)
        def _():
            pltpu.async_copy(row_hbm.at[wid, j0 + 2], ridx.at[0], si0)

        pltpu.make_async_copy(row_hbm.at[wid, j0 + 1], ridx.at[1], si1).wait()
        pltpu.async_copy(g_hbm.at[ridx.at[1]], rows1, sg1)
        pltpu.sync_copy(rows0, acc.at[cidx.at[j0]], add=True)

        pltpu.make_async_copy(g_hbm.at[ridx.at[1]], rows1, sg1).wait()

        @pl.when(j0 + 2 < NCHUNK)
        def _():
            pltpu.async_copy(row_hbm.at[wid, j0 + 3], ridx.at[1], si1)
            pltpu.make_async_copy(row_hbm.at[wid, j0 + 2], ridx.at[0], si0).wait()
            pltpu.async_copy(g_hbm.at[ridx.at[0]], rows0, sg0)

        pltpu.sync_copy(rows1, acc.at[cidx.at[j0 + 1]], add=True)
        return carry

    lax.fori_loop(0, NCHUNK // 2, body, 0)
    plsc.subcore_barrier()

    def obody(t, carry):
        r0 = sid * NPT + t * RB
        pltpu.sync_copy(acc.at[pl.ds(r0, RB)], rows0)
        pltpu.sync_copy(rows0, out_hbm.at[cid, pl.ds(r0, RB)])
        return carry

    lax.fori_loop(0, NPT // RB, obody, 0)


_agg_kernel = functools.partial(
    pl.kernel,
    out_type=jax.ShapeDtypeStruct((NC, N_PAD, D), jnp.float32),
    mesh=_sc_mesh,
    scratch_types=[
        pltpu.VMEM((2, CH), jnp.int32),
        pltpu.VMEM((NCHUNK, CH), jnp.int32),
        pltpu.VMEM((CH, D), jnp.float32),
        pltpu.VMEM((CH, D), jnp.float32),
        pltpu.VMEM_SHARED((N_PAD, D), jnp.float32),
        pltpu.SemaphoreType.DMA,
        pltpu.SemaphoreType.DMA,
        pltpu.SemaphoreType.DMA,
        pltpu.SemaphoreType.DMA,
    ],
)(_agg_body)


def _epi_body(a_ref, dp_ref, g_ref, b_ref, out_ref):
    s = a_ref[0] + a_ref[1] + g_ref[...]
    out_ref[...] = jnp.maximum(s * _dis_col(dp_ref) + b_ref[...], 0.0)


_epi_kernel = pl.pallas_call(
    _epi_body,
    grid=(N_PAD // RB,),
    in_specs=[
        pl.BlockSpec((2, RB, D), lambda i: (0, i, 0)),
        pl.BlockSpec((2, N_PAD // 128, 128), lambda i: (0, 0, 0)),
        pl.BlockSpec((RB, D), lambda i: (i, 0)),
        pl.BlockSpec((1, D), lambda i: (0, 0)),
    ],
    out_specs=pl.BlockSpec((RB, D), lambda i: (i, 0)),
    out_shape=jax.ShapeDtypeStruct((N_PAD, D), jnp.float32),
)


def kernel(node_feature, edge_index, W, b):
    row = edge_index[0]
    col = edge_index[1]
    pad_e = E_PAD - E
    row3 = jnp.concatenate(
        [row, jnp.zeros((pad_e,), jnp.int32)]).reshape(NW, NCHUNK, CH)
    col3 = jnp.concatenate(
        [col, jnp.full((pad_e,), N, jnp.int32)]).reshape(NW, NCHUNK, CH)
    x_pad = jnp.pad(node_feature, ((0, N_PAD - N), (0, 0)))
    zeros1 = jnp.zeros((N_PAD,), jnp.float32)
    zeros2 = jnp.zeros((N_PAD, D), jnp.float32)

    dp = _deg_kernel(col3, zeros1)
    dp3 = dp.reshape(2, N_PAD // 128, 128)
    g = _mm_kernel(x_pad, W, dp3)
    a = _agg_kernel(row3, col3, g, zeros2)
    out = _epi_kernel(a, dp3, g, b.reshape(1, D))
    return out[:N]


# speedup vs baseline: 1.4596x; 1.4596x over previous
import functools

import jax
import jax.numpy as jnp
from jax import lax
from jax.experimental import pallas as pl
from jax.experimental.pallas import tpu as pltpu
from jax.experimental.pallas import tpu_sc as plsc

N = 10000
E = 320000
D = 128
NC = 2
NS = 16
NW = NC * NS
CH = 64
NCHUNK = 2 * (-(-E // (NW * CH * 2)))
EPT = NCHUNK * CH
E_PAD = EPT * NW
N_PAD = 10240
NPT = N_PAD // NS
RB = 128

_sc_mesh = plsc.VectorSubcoreMesh(core_axis_name="c", subcore_axis_name="s")


def _deg_body(col_hbm, zeros_hbm, out_hbm, cidx, ones, tmp, shared):
    cid = lax.axis_index("c")
    sid = lax.axis_index("s")
    wid = sid * NC + cid
    pltpu.sync_copy(zeros_hbm.at[pl.ds(sid * NPT, NPT)], tmp)
    pltpu.sync_copy(tmp, shared.at[pl.ds(sid * NPT, NPT)])
    for i in range(CH // 16):
        ones[pl.ds(i * 16, 16)] = jnp.ones((16,), jnp.float32)
    pltpu.sync_copy(col_hbm.at[wid], cidx)
    plsc.subcore_barrier()

    def body(j, carry):
        pltpu.sync_copy(ones, shared.at[cidx.at[pl.ds(j * CH, CH)]], add=True)
        return carry

    lax.fori_loop(0, NCHUNK, body, 0)
    plsc.subcore_barrier()
    pltpu.sync_copy(shared.at[pl.ds(sid * NPT, NPT)], tmp)
    pltpu.sync_copy(tmp, out_hbm.at[cid, pl.ds(sid * NPT, NPT)])


_deg_kernel = functools.partial(
    pl.kernel,
    out_type=jax.ShapeDtypeStruct((NC, N_PAD), jnp.float32),
    mesh=_sc_mesh,
    scratch_types=[
        pltpu.VMEM((EPT,), jnp.int32),
        pltpu.VMEM((CH,), jnp.float32),
        pltpu.VMEM((NPT,), jnp.float32),
        pltpu.VMEM_SHARED((N_PAD,), jnp.float32),
    ],
)(_deg_body)


def _dis_col(dp_ref):
    pid = pl.program_id(0)
    deg_row = dp_ref[0, pid, :] + dp_ref[1, pid, :] + 1.0
    dis_row = lax.rsqrt(deg_row)
    m = (lax.broadcasted_iota(jnp.int32, (RB, RB), 0)
         == lax.broadcasted_iota(jnp.int32, (RB, RB), 1))
    return jnp.sum(jnp.where(m, dis_row[None, :], 0.0), axis=1,
                   keepdims=True)


def _mm_body(x_ref, w_ref, dp_ref, g_ref):
    h = jnp.dot(x_ref[...], w_ref[...], preferred_element_type=jnp.float32)
    g_ref[...] = h * _dis_col(dp_ref)


_mm_kernel = pl.pallas_call(
    _mm_body,
    grid=(N_PAD // RB,),
    in_specs=[
        pl.BlockSpec((RB, D), lambda i: (i, 0)),
        pl.BlockSpec((D, D), lambda i: (0, 0)),
        pl.BlockSpec((2, N_PAD // 128, 128), lambda i: (0, 0, 0)),
    ],
    out_specs=pl.BlockSpec((RB, D), lambda i: (i, 0)),
    out_shape=jax.ShapeDtypeStruct((N_PAD, D), jnp.float32),
)


def _agg_body(row_hbm, col_hbm, g_hbm, zeros_hbm, out_hbm,
              ridx, cidx, rows0, rows1, acc, sg0, sg1):
    cid = lax.axis_index("c")
    sid = lax.axis_index("s")
    wid = sid * NC + cid

    pltpu.sync_copy(zeros_hbm.at[pl.ds(sid * NPT, NPT)],
                    acc.at[pl.ds(sid * NPT, NPT)])
    pltpu.sync_copy(row_hbm.at[wid], ridx)
    pltpu.sync_copy(col_hbm.at[wid], cidx)
    plsc.subcore_barrier()

    def rix(j):
        return ridx.at[pl.ds(j * CH, CH)]

    def cix(j):
        return cidx.at[pl.ds(j * CH, CH)]

    pltpu.async_copy(g_hbm.at[rix(0)], rows0, sg0)

    def body(t, carry):
        j0 = 2 * t
        pltpu.async_copy(g_hbm.at[rix(j0 + 1)], rows1, sg1)
        pltpu.make_async_copy(g_hbm.at[rix(j0)], rows0, sg0).wait()
        pltpu.sync_copy(rows0, acc.at[cix(j0)], add=True)

        @pl.when(j0 + 2 < NCHUNK)
        def _():
            pltpu.async_copy(g_hbm.at[rix(j0 + 2)], rows0, sg0)

        pltpu.make_async_copy(g_hbm.at[rix(j0 + 1)], rows1, sg1).wait()
        pltpu.sync_copy(rows1, acc.at[cix(j0 + 1)], add=True)
        return carry

    lax.fori_loop(0, NCHUNK // 2, body, 0)
    plsc.subcore_barrier()

    pltpu.sync_copy(acc.at[pl.ds(sid * NPT, NPT)],
                    out_hbm.at[cid, pl.ds(sid * NPT, NPT)])


_agg_kernel = functools.partial(
    pl.kernel,
    out_type=jax.ShapeDtypeStruct((NC, N_PAD, D), jnp.float32),
    mesh=_sc_mesh,
    scratch_types=[
        pltpu.VMEM((EPT,), jnp.int32),
        pltpu.VMEM((EPT,), jnp.int32),
        pltpu.VMEM((CH, D), jnp.float32),
        pltpu.VMEM((CH, D), jnp.float32),
        pltpu.VMEM_SHARED((N_PAD, D), jnp.float32),
        pltpu.SemaphoreType.DMA,
        pltpu.SemaphoreType.DMA,
    ],
)(_agg_body)


def _epi_body(a_ref, dp_ref, g_ref, b_ref, out_ref):
    s = a_ref[0] + a_ref[1] + g_ref[...]
    out_ref[...] = jnp.maximum(s * _dis_col(dp_ref) + b_ref[...], 0.0)


_epi_kernel = pl.pallas_call(
    _epi_body,
    grid=(N_PAD // RB,),
    in_specs=[
        pl.BlockSpec((2, RB, D), lambda i: (0, i, 0)),
        pl.BlockSpec((2, N_PAD // 128, 128), lambda i: (0, 0, 0)),
        pl.BlockSpec((RB, D), lambda i: (i, 0)),
        pl.BlockSpec((1, D), lambda i: (0, 0)),
    ],
    out_specs=pl.BlockSpec((RB, D), lambda i: (i, 0)),
    out_shape=jax.ShapeDtypeStruct((N_PAD, D), jnp.float32),
)


def kernel(node_feature, edge_index, W, b):
    row = edge_index[0]
    col = edge_index[1]
    pad_e = E_PAD - E
    row3 = jnp.concatenate(
        [row, jnp.zeros((pad_e,), jnp.int32)]).reshape(NW, EPT)
    col3 = jnp.concatenate(
        [col, jnp.full((pad_e,), N, jnp.int32)]).reshape(NW, EPT)
    x_pad = jnp.pad(node_feature, ((0, N_PAD - N), (0, 0)))
    zeros1 = jnp.zeros((N_PAD,), jnp.float32)
    zeros2 = jnp.zeros((N_PAD, D), jnp.float32)

    dp = _deg_kernel(col3, zeros1)
    dp3 = dp.reshape(2, N_PAD // 128, 128)
    g = _mm_kernel(x_pad, W, dp3)
    a = _agg_kernel(row3, col3, g, zeros2)
    out = _epi_kernel(a, dp3, g, b.reshape(1, D))
    return out[:N]
